# baseline (speedup 1.0000x reference)
import jax
import jax.numpy as jnp
from jax.experimental import pallas as pl
from jax.experimental.pallas import tpu as pltpu

NUM_EMB_ = 1024
DIM_ = 64
CC_ = 0.6
TOK_BLOCK = 2048


def _vq_block_kernel(x_ref, emb_ref, w_ref, e2_ref, q_ref, idx_ref, ps_ref):
    x = x_ref[...]
    emb = emb_ref[...]
    w = w_ref[...]
    e2 = e2_ref[...]
    dot = jax.lax.dot_general(x, emb, (((1,), (1,)), ((), ())),
                              preferred_element_type=jnp.float32)
    x2 = jnp.sum(x * x, axis=1, keepdims=True)
    dist = (x2 + e2 - 2.0 * dot) * w
    m = jnp.min(dist, axis=1, keepdims=True)
    kio = jax.lax.broadcasted_iota(jnp.int32, dist.shape, 1)
    idx = jnp.min(jnp.where(dist == m, kio, NUM_EMB_), axis=1)
    oh = (kio == idx[:, None]).astype(jnp.float32)
    q = jax.lax.dot_general(oh, emb, (((1,), (0,)), ((), ())),
                            preferred_element_type=jnp.float32)
    q_ref[...] = q
    idx_ref[0, 0, :] = idx
    d = q - x
    ps_ref[...] = jnp.full((1, 1, 128), jnp.sum(d * d), dtype=jnp.float32)


def kernel(inputs, emb_weight, scaling):
    B, S, D = inputs.shape
    K = emb_weight.shape[0]
    N = B * S
    G = N // TOK_BLOCK
    flat = inputs.reshape(N, D)
    hr_values = jnp.linspace(40.0, 180.0, K)
    w = (1.0 + scaling * ((hr_values - 100.0) / 70.0)).reshape(1, K)
    e2 = jnp.sum(emb_weight ** 2, axis=1).reshape(1, K)

    q, idx3, ps = pl.pallas_call(
        _vq_block_kernel,
        grid=(G,),
        in_specs=[
            pl.BlockSpec((TOK_BLOCK, D), lambda i: (i, 0)),
            pl.BlockSpec((K, D), lambda i: (0, 0)),
            pl.BlockSpec((1, K), lambda i: (0, 0)),
            pl.BlockSpec((1, K), lambda i: (0, 0)),
        ],
        out_specs=[
            pl.BlockSpec((TOK_BLOCK, D), lambda i: (i, 0)),
            pl.BlockSpec((1, 1, TOK_BLOCK), lambda i: (i, 0, 0)),
            pl.BlockSpec((1, 1, 128), lambda i: (i, 0, 0)),
        ],
        out_shape=[
            jax.ShapeDtypeStruct((N, D), jnp.float32),
            jax.ShapeDtypeStruct((G, 1, TOK_BLOCK), jnp.int32),
            jax.ShapeDtypeStruct((G, 1, 128), jnp.float32),
        ],
        compiler_params=pltpu.CompilerParams(
            dimension_semantics=("parallel",),
        ),
    )(flat, emb_weight, w, e2)

    loss = (1.0 + CC_) * jnp.sum(ps[:, 0, 0]) / (N * D)
    quantized_st = q.reshape(inputs.shape)
    encoding_indices = idx3.reshape(B, S)
    return (quantized_st, loss, encoding_indices)


# speedup vs baseline: 1.0532x; 1.0532x over previous
import jax
import jax.numpy as jnp
from jax.experimental import pallas as pl
from jax.experimental.pallas import tpu as pltpu

NUM_EMB_ = 1024
DIM_ = 64
CC_ = 0.6
BATCH_BLOCK = 4


def _vq_block_kernel(x_ref, emb_ref, w_ref, e2_ref, q_ref, idx_ref, ps_ref):
    x = x_ref[...]
    emb = emb_ref[...]
    w = w_ref[...]
    e2 = e2_ref[...]
    dot = jax.lax.dot_general(x, emb, (((2,), (1,)), ((), ())),
                              preferred_element_type=jnp.float32)
    x2 = jnp.sum(x * x, axis=2, keepdims=True)
    dist = (x2 + e2 - 2.0 * dot) * w
    m = jnp.min(dist, axis=2, keepdims=True)
    kio = jax.lax.broadcasted_iota(jnp.int32, dist.shape, 2)
    idx = jnp.min(jnp.where(dist == m, kio, NUM_EMB_), axis=2)
    oh = (kio == idx[:, :, None]).astype(jnp.float32)
    q = jax.lax.dot_general(oh, emb, (((2,), (0,)), ((), ())),
                            preferred_element_type=jnp.float32)
    q_ref[...] = q
    idx_ref[0] = idx
    d = q - x
    ps_ref[...] = jnp.full((1, 1, 128), jnp.sum(d * d), dtype=jnp.float32)


def kernel(inputs, emb_weight, scaling):
    B, S, D = inputs.shape
    K = emb_weight.shape[0]
    N = B * S
    G = B // BATCH_BLOCK
    hr_values = jnp.linspace(40.0, 180.0, K)
    w = (1.0 + scaling * ((hr_values - 100.0) / 70.0)).reshape(1, 1, K)
    e2 = jnp.sum(emb_weight ** 2, axis=1).reshape(1, 1, K)

    q, idx4, ps = pl.pallas_call(
        _vq_block_kernel,
        grid=(G,),
        in_specs=[
            pl.BlockSpec((BATCH_BLOCK, S, D), lambda i: (i, 0, 0)),
            pl.BlockSpec((K, D), lambda i: (0, 0)),
            pl.BlockSpec((1, 1, K), lambda i: (0, 0, 0)),
            pl.BlockSpec((1, 1, K), lambda i: (0, 0, 0)),
        ],
        out_specs=[
            pl.BlockSpec((BATCH_BLOCK, S, D), lambda i: (i, 0, 0)),
            pl.BlockSpec((1, BATCH_BLOCK, S), lambda i: (i, 0, 0)),
            pl.BlockSpec((1, 1, 128), lambda i: (i, 0, 0)),
        ],
        out_shape=[
            jax.ShapeDtypeStruct((B, S, D), jnp.float32),
            jax.ShapeDtypeStruct((G, BATCH_BLOCK, S), jnp.int32),
            jax.ShapeDtypeStruct((G, 1, 128), jnp.float32),
        ],
        compiler_params=pltpu.CompilerParams(
            dimension_semantics=("arbitrary",),
        ),
    )(inputs, emb_weight, w, e2)

    loss = (1.0 + CC_) * jnp.sum(ps[:, 0, 0]) / (N * D)
    encoding_indices = idx4.reshape(B, S)
    return (q, loss, encoding_indices)


# speedup vs baseline: 1.5915x; 1.5111x over previous
import jax
import jax.numpy as jnp
from jax.experimental import pallas as pl
from jax.experimental.pallas import tpu as pltpu

NUM_EMB_ = 1024
DIM_ = 64
CC_ = 0.6
BATCH_BLOCK = 8


def _vq_block_kernel(xt_ref, embt_ref, w_ref, e2_ref, qt_ref, idx_ref, ps_ref):
    embt = embt_ref[...]
    w = w_ref[...]
    e2 = e2_ref[...]
    acc = jnp.float32(0.0)
    for b in range(BATCH_BLOCK):
        xb = xt_ref[b]
        dT = jax.lax.dot_general(embt, xb, (((0,), (0,)), ((), ())),
                                 preferred_element_type=jnp.float32)
        x2 = jnp.sum(xb * xb, axis=0, keepdims=True)
        dist = (x2 + e2 - 2.0 * dT) * w
        m = jnp.min(dist, axis=0, keepdims=True)
        kio = jax.lax.broadcasted_iota(jnp.int32, dist.shape, 0)
        idx = jnp.min(jnp.where(dist == m, kio, NUM_EMB_), axis=0)
        oh = (kio == idx[None, :]).astype(jnp.float32)
        qt = jax.lax.dot_general(embt, oh, (((1,), (0,)), ((), ())),
                                 preferred_element_type=jnp.float32)
        qt_ref[b] = qt
        idx_ref[b] = idx
        d = qt - xb
        acc = acc + jnp.sum(d * d)
    ps_ref[...] = jnp.full((1, 1, 128), acc, dtype=jnp.float32)


def kernel(inputs, emb_weight, scaling):
    B, S, D = inputs.shape
    K = emb_weight.shape[0]
    N = B * S
    G = B // BATCH_BLOCK
    xt = jnp.swapaxes(inputs, 1, 2)
    embt = emb_weight.T
    hr_values = jnp.linspace(40.0, 180.0, K)
    w = (1.0 + scaling * ((hr_values - 100.0) / 70.0)).reshape(K, 1)
    e2 = jnp.sum(emb_weight ** 2, axis=1).reshape(K, 1)

    qt, idx, ps = pl.pallas_call(
        _vq_block_kernel,
        grid=(G,),
        in_specs=[
            pl.BlockSpec((BATCH_BLOCK, D, S), lambda i: (i, 0, 0)),
            pl.BlockSpec((D, K), lambda i: (0, 0)),
            pl.BlockSpec((K, 1), lambda i: (0, 0)),
            pl.BlockSpec((K, 1), lambda i: (0, 0)),
        ],
        out_specs=[
            pl.BlockSpec((BATCH_BLOCK, D, S), lambda i: (i, 0, 0)),
            pl.BlockSpec((BATCH_BLOCK, S), lambda i: (i, 0)),
            pl.BlockSpec((1, 1, 128), lambda i: (i, 0, 0)),
        ],
        out_shape=[
            jax.ShapeDtypeStruct((B, D, S), jnp.float32),
            jax.ShapeDtypeStruct((B, S), jnp.int32),
            jax.ShapeDtypeStruct((G, 1, 128), jnp.float32),
        ],
        compiler_params=pltpu.CompilerParams(
            dimension_semantics=("arbitrary",),
        ),
    )(xt, embt, w, e2)

    loss = (1.0 + CC_) * jnp.sum(ps[:, 0, 0]) / (N * D)
    quantized_st = jnp.swapaxes(qt, 1, 2)
    return (quantized_st, loss, idx)


# speedup vs baseline: 2.0837x; 1.3093x over previous
import jax
import jax.numpy as jnp
from jax.experimental import pallas as pl
from jax.experimental.pallas import tpu as pltpu

NUM_EMB_ = 1024
DIM_ = 64
CC_ = 0.6
BATCH_BLOCK = 8


def _vq_block_kernel(xt_ref, embt_ref, w_ref, e2_ref, qt_ref, idx_ref, ps_ref):
    embt = embt_ref[...]
    w = w_ref[...]
    e2 = e2_ref[...]
    acc = jnp.float32(0.0)
    for b in range(BATCH_BLOCK):
        xb = xt_ref[b]
        dT = jax.lax.dot_general(embt, xb, (((0,), (0,)), ((), ())),
                                 preferred_element_type=jnp.float32)
        x2 = jnp.sum(xb * xb, axis=0, keepdims=True)
        dist = (x2 + e2 - 2.0 * dT) * w
        idx = jnp.argmin(dist, axis=0)
        kio = jax.lax.broadcasted_iota(jnp.int32, dist.shape, 0)
        oh = (kio == idx[None, :]).astype(jnp.float32)
        qt = jax.lax.dot_general(embt, oh, (((1,), (0,)), ((), ())),
                                 preferred_element_type=jnp.float32)
        qt_ref[b] = qt
        idx_ref[b] = idx
        d = qt - xb
        acc = acc + jnp.sum(d * d)
    ps_ref[...] = jnp.full((1, 1, 128), acc, dtype=jnp.float32)


def kernel(inputs, emb_weight, scaling):
    B, S, D = inputs.shape
    K = emb_weight.shape[0]
    N = B * S
    G = B // BATCH_BLOCK
    xt = jnp.swapaxes(inputs, 1, 2)
    embt = emb_weight.T
    hr_values = jnp.linspace(40.0, 180.0, K)
    w = (1.0 + scaling * ((hr_values - 100.0) / 70.0)).reshape(K, 1)
    e2 = jnp.sum(emb_weight ** 2, axis=1).reshape(K, 1)

    qt, idx, ps = pl.pallas_call(
        _vq_block_kernel,
        grid=(G,),
        in_specs=[
            pl.BlockSpec((BATCH_BLOCK, D, S), lambda i: (i, 0, 0)),
            pl.BlockSpec((D, K), lambda i: (0, 0)),
            pl.BlockSpec((K, 1), lambda i: (0, 0)),
            pl.BlockSpec((K, 1), lambda i: (0, 0)),
        ],
        out_specs=[
            pl.BlockSpec((BATCH_BLOCK, D, S), lambda i: (i, 0, 0)),
            pl.BlockSpec((BATCH_BLOCK, S), lambda i: (i, 0)),
            pl.BlockSpec((1, 1, 128), lambda i: (i, 0, 0)),
        ],
        out_shape=[
            jax.ShapeDtypeStruct((B, D, S), jnp.float32),
            jax.ShapeDtypeStruct((B, S), jnp.int32),
            jax.ShapeDtypeStruct((G, 1, 128), jnp.float32),
        ],
        compiler_params=pltpu.CompilerParams(
            dimension_semantics=("arbitrary",),
        ),
    )(xt, embt, w, e2)

    loss = (1.0 + CC_) * jnp.sum(ps[:, 0, 0]) / (N * D)
    quantized_st = jnp.swapaxes(qt, 1, 2)
    return (quantized_st, loss, idx)


# speedup vs baseline: 2.4019x; 1.1527x over previous
import jax
import jax.numpy as jnp
from jax.experimental import pallas as pl
from jax.experimental.pallas import tpu as pltpu

NUM_EMB_ = 1024
DIM_ = 64
CC_ = 0.6
BATCH_BLOCK = 8


def _vq_block_kernel(xt_ref, embt_ref, w_ref, qt_ref, idx_ref, loss_ref):
    i = pl.program_id(0)
    ng = pl.num_programs(0)
    embt = embt_ref[...]
    w = w_ref[...]
    e2 = jnp.sum(embt * embt, axis=0, keepdims=True).reshape(NUM_EMB_, 1)
    acc = jnp.float32(0.0)
    for b in range(BATCH_BLOCK):
        xb = xt_ref[b]
        dT = jax.lax.dot_general(embt, xb, (((0,), (0,)), ((), ())),
                                 preferred_element_type=jnp.float32)
        x2 = jnp.sum(xb * xb, axis=0, keepdims=True)
        dist = (x2 + e2 - 2.0 * dT) * w
        idx = jnp.argmin(dist, axis=0)
        kio = jax.lax.broadcasted_iota(jnp.int32, dist.shape, 0)
        oh = (kio == idx[None, :]).astype(jnp.float32)
        qt = jax.lax.dot_general(embt, oh, (((1,), (0,)), ((), ())),
                                 preferred_element_type=jnp.float32)
        qt_ref[b] = qt
        idx_ref[b] = idx
        d = qt - xb
        acc = acc + jnp.sum(d * d)

    @pl.when(i == 0)
    def _init():
        loss_ref[...] = jnp.zeros((1, 1), jnp.float32)

    loss_ref[...] += jnp.full((1, 1), acc, jnp.float32)

    @pl.when(i == ng - 1)
    def _finalize():
        loss_ref[...] = loss_ref[...] * ((1.0 + CC_) / (BATCH_BLOCK * ng * 576 * DIM_))


def kernel(inputs, emb_weight, scaling):
    B, S, D = inputs.shape
    K = emb_weight.shape[0]
    G = B // BATCH_BLOCK
    xt = jnp.swapaxes(inputs, 1, 2)
    embt = emb_weight.T
    hr_values = jnp.linspace(40.0, 180.0, K)
    w = (1.0 + scaling * ((hr_values - 100.0) / 70.0)).reshape(K, 1)

    qt, idx, loss2 = pl.pallas_call(
        _vq_block_kernel,
        grid=(G,),
        in_specs=[
            pl.BlockSpec((BATCH_BLOCK, D, S), lambda i: (i, 0, 0)),
            pl.BlockSpec((D, K), lambda i: (0, 0)),
            pl.BlockSpec((K, 1), lambda i: (0, 0)),
        ],
        out_specs=[
            pl.BlockSpec((BATCH_BLOCK, D, S), lambda i: (i, 0, 0)),
            pl.BlockSpec((BATCH_BLOCK, S), lambda i: (i, 0)),
            pl.BlockSpec((1, 1), lambda i: (0, 0)),
        ],
        out_shape=[
            jax.ShapeDtypeStruct((B, D, S), jnp.float32),
            jax.ShapeDtypeStruct((B, S), jnp.int32),
            jax.ShapeDtypeStruct((1, 1), jnp.float32),
        ],
        compiler_params=pltpu.CompilerParams(
            dimension_semantics=("arbitrary",),
        ),
    )(xt, embt, w)

    loss = loss2[0, 0]
    quantized_st = jnp.swapaxes(qt, 1, 2)
    return (quantized_st, loss, idx)


# speedup vs baseline: 2.4327x; 1.0128x over previous
import jax
import jax.numpy as jnp
from jax.experimental import pallas as pl
from jax.experimental.pallas import tpu as pltpu

NUM_EMB_ = 1024
DIM_ = 64
CC_ = 0.6
BATCH_BLOCK = 16


def _vq_block_kernel(xt_ref, embt_ref, w_ref, qt_ref, idx_ref, loss_ref):
    i = pl.program_id(0)
    ng = pl.num_programs(0)
    embt = embt_ref[...]
    w = w_ref[...]
    e2 = jnp.sum(embt * embt, axis=0, keepdims=True).reshape(NUM_EMB_, 1)
    acc = jnp.float32(0.0)
    for b in range(BATCH_BLOCK):
        xb = xt_ref[b]
        dT = jax.lax.dot_general(embt, xb, (((0,), (0,)), ((), ())),
                                 preferred_element_type=jnp.float32)
        x2 = jnp.sum(xb * xb, axis=0, keepdims=True)
        dist = (x2 + e2 - 2.0 * dT) * w
        idx = jnp.argmin(dist, axis=0)
        kio = jax.lax.broadcasted_iota(jnp.int32, dist.shape, 0)
        oh = (kio == idx[None, :]).astype(jnp.float32)
        qt = jax.lax.dot_general(embt, oh, (((1,), (0,)), ((), ())),
                                 preferred_element_type=jnp.float32)
        qt_ref[b] = qt
        idx_ref[b] = idx
        d = qt - xb
        acc = acc + jnp.sum(d * d)

    @pl.when(i == 0)
    def _init():
        loss_ref[...] = jnp.zeros((1, 1), jnp.float32)

    loss_ref[...] += jnp.full((1, 1), acc, jnp.float32)

    @pl.when(i == ng - 1)
    def _finalize():
        loss_ref[...] = loss_ref[...] * ((1.0 + CC_) / (BATCH_BLOCK * ng * 576 * DIM_))


def kernel(inputs, emb_weight, scaling):
    B, S, D = inputs.shape
    K = emb_weight.shape[0]
    G = B // BATCH_BLOCK
    xt = jnp.swapaxes(inputs, 1, 2)
    embt = emb_weight.T
    hr_values = jnp.linspace(40.0, 180.0, K)
    w = (1.0 + scaling * ((hr_values - 100.0) / 70.0)).reshape(K, 1)

    qt, idx, loss2 = pl.pallas_call(
        _vq_block_kernel,
        grid=(G,),
        in_specs=[
            pl.BlockSpec((BATCH_BLOCK, D, S), lambda i: (i, 0, 0)),
            pl.BlockSpec((D, K), lambda i: (0, 0)),
            pl.BlockSpec((K, 1), lambda i: (0, 0)),
        ],
        out_specs=[
            pl.BlockSpec((BATCH_BLOCK, D, S), lambda i: (i, 0, 0)),
            pl.BlockSpec((BATCH_BLOCK, S), lambda i: (i, 0)),
            pl.BlockSpec((1, 1), lambda i: (0, 0)),
        ],
        out_shape=[
            jax.ShapeDtypeStruct((B, D, S), jnp.float32),
            jax.ShapeDtypeStruct((B, S), jnp.int32),
            jax.ShapeDtypeStruct((1, 1), jnp.float32),
        ],
        compiler_params=pltpu.CompilerParams(
            dimension_semantics=("arbitrary",),
        ),
    )(xt, embt, w)

    loss = loss2[0, 0]
    quantized_st = jnp.swapaxes(qt, 1, 2)
    return (quantized_st, loss, idx)
